# Initial kernel scaffold; baseline (speedup 1.0000x reference)
import jax
import jax.numpy as jnp
from jax.experimental import pallas as pl


def kernel(input_tensor, hidden, memory, weight_ih, weight_hh, bias_ih, bias_hh, W_out, b_out):
    raise NotImplementedError("write your pallas kernel here")


# speedup vs baseline: 2.3417x; 2.3417x over previous
import functools

import jax
import jax.numpy as jnp
from jax import lax
from jax.experimental import pallas as pl
from jax.experimental.pallas import tpu as pltpu
from jax.experimental.pallas import tpu_sc as plsc

B = 16384
IN = 130
H = 128
SW = 2
GRID = 600
GX = GRID + 3 * SW
GY = GRID + 3 * SW
NCELL = GX * GY
NOFF = 2 * SW + 1
K = NOFF * NOFF

NW = 32
RPW = B * K // NW
CH = 128
NCH = RPW // CH
SPW = B // NW
NSC = SPW // CH

BB = 512


def _dense_body(feat_ref, hid_ref, ctx_ref, wih_ref, whh_ref, bih_ref,
                bhh_ref, wout_ref, bout_ref, out_ref, upd_ref):
    feat = feat_ref[...]
    hid = hid_ref[...]
    dn = (((1,), (1,)), ((), ()))
    gi = lax.dot_general(feat, wih_ref[...], dn,
                         preferred_element_type=jnp.float32) + bih_ref[...]
    gh = lax.dot_general(hid, whh_ref[...], dn,
                         preferred_element_type=jnp.float32) + bhh_ref[...]
    resetgate = jax.nn.sigmoid(gi[:, 0:H] + gh[:, 0:H])
    updategate = jax.nn.sigmoid(gi[:, H:2 * H] + gh[:, H:2 * H])
    spatialgate = jax.nn.sigmoid(gi[:, 3 * H:4 * H] + gh[:, 3 * H:4 * H])
    newgate = jnp.tanh(gi[:, 2 * H:3 * H] + resetgate * gh[:, 2 * H:3 * H])

    ctx = ctx_ref[...]
    a = jnp.sum(ctx * newgate[:, None, :], axis=2)
    zero_mask = a == 0.0
    am = jnp.where(zero_mask, -jnp.inf, a)
    m = jnp.max(am, axis=1, keepdims=True)
    m_safe = jnp.where(jnp.isfinite(m), m, 0.0)
    e = jnp.where(zero_mask, 0.0, jnp.exp(a - m_safe))
    s = jnp.sum(e, axis=1, keepdims=True)
    attn = jnp.where(s > 0.0, e / jnp.where(s > 0.0, s, 1.0), 0.0)
    mix = jnp.sum(attn[:, :, None] * ctx, axis=1)

    wout = wout_ref[...]
    pre = (lax.dot_general(mix, wout[:, 0:H], dn,
                           preferred_element_type=jnp.float32)
           + lax.dot_general(newgate, wout[:, H:2 * H], dn,
                             preferred_element_type=jnp.float32)
           + bout_ref[...])
    atten_cs = jnp.tanh(pre)
    curr = newgate + spatialgate * atten_cs
    out = curr + updategate * (hid - curr)
    read = ctx[:, K // 2, :]
    out_ref[...] = out
    upd_ref[...] = spatialgate * read + (1.0 - spatialgate) * out


def _dense(feature, hidden, ctx, wih, whh, bih, bhh, wout, bout):
    grid = (B // BB,)
    bs_b = pl.BlockSpec((BB, H), lambda i: (i, 0))
    bs_ctx = pl.BlockSpec((BB, K, H), lambda i: (i, 0, 0))
    full = lambda shape: pl.BlockSpec(shape, lambda i: tuple(0 for _ in shape))
    return pl.pallas_call(
        _dense_body,
        grid=grid,
        in_specs=[bs_b, bs_b, bs_ctx, full((4 * H, H)), full((4 * H, H)),
                  full((1, 4 * H)), full((1, 4 * H)), full((H, 2 * H)),
                  full((1, H))],
        out_specs=[bs_b, bs_b],
        out_shape=[jax.ShapeDtypeStruct((B, H), jnp.float32),
                   jax.ShapeDtypeStruct((B, H), jnp.float32)],
        compiler_params=pltpu.CompilerParams(
            dimension_semantics=("arbitrary",)),
    )(feature, hidden, ctx, wih, whh, bih.reshape(1, 4 * H),
      bhh.reshape(1, 4 * H), wout, bout.reshape(1, H))


_SC_MESH = plsc.VectorSubcoreMesh(core_axis_name="c", subcore_axis_name="s")


@functools.partial(
    pl.kernel,
    out_type=jax.ShapeDtypeStruct((B * K, H), jnp.float32),
    mesh=_SC_MESH,
    scratch_types=[
        pltpu.VMEM((NCH, CH), jnp.int32),
        pltpu.VMEM((2, CH, H), jnp.float32),
        pltpu.SemaphoreType.DMA,
        pltpu.SemaphoreType.DMA,
    ],
)
def _sc_gather(mem_ref, idx_ref, ctx_ref, idx_v, rows_v, semA, semB):
    wid = lax.axis_index("s") * 2 + lax.axis_index("c")
    base = wid * RPW
    pltpu.sync_copy(idx_ref.at[wid], idx_v)
    sems = (semA, semB)

    def start(j, slot):
        pltpu.make_async_copy(mem_ref.at[idx_v.at[j]], rows_v.at[slot],
                              sems[slot]).start()

    def finish(j, slot):
        pltpu.make_async_copy(mem_ref.at[idx_v.at[j]], rows_v.at[slot],
                              sems[slot]).wait()
        pltpu.sync_copy(rows_v.at[slot], ctx_ref.at[pl.ds(base + j * CH, CH)])

    start(0, 0)
    start(1, 1)

    @pl.loop(0, NCH, step=2)
    def _(g):
        for b in range(2):
            j = g + b
            finish(j, b)

            @pl.when(j + 2 < NCH)
            def _():
                start(j + 2, b)


@functools.partial(
    pl.kernel,
    out_type=(),
    mesh=_SC_MESH,
    scratch_types=[
        pltpu.VMEM((NSC, CH), jnp.int32),
        pltpu.VMEM((NSC, CH), jnp.int32),
        pltpu.VMEM((CH, H), jnp.float32),
        pltpu.SemaphoreType.DMA,
    ],
)
def _sc_scatter(upd_ref, win_ref, cell_ref, mem_ref, win_v, cell_v, rows_v,
                sem):
    wid = lax.axis_index("s") * 2 + lax.axis_index("c")
    pltpu.sync_copy(win_ref.at[wid], win_v)
    pltpu.sync_copy(cell_ref.at[wid], cell_v)
    for j in range(NSC):
        cp = pltpu.make_async_copy(upd_ref.at[win_v.at[j]], rows_v, sem)
        cp.start()
        cp.wait()
        pltpu.sync_copy(rows_v, mem_ref.at[cell_v.at[j]])


def kernel(input_tensor, hidden, memory, weight_ih, weight_hh, bias_ih,
           bias_hh, W_out, b_out):
    feature = input_tensor[:, :H]
    coords = input_tensor[:, H:].astype(jnp.int32) + SW
    gx = jnp.clip(coords[:, 0], 0, GX - 1)
    gy = jnp.clip(coords[:, 1], 0, GY - 1)
    offs = jnp.arange(-SW, SW + 1, dtype=jnp.int32)
    x_idx = jnp.clip(gx[:, None] + offs[None, :], 0, GX - 1)
    y_idx = jnp.clip(gy[:, None] + offs[None, :], 0, GY - 1)
    idx25 = (x_idx[:, :, None] * GY + y_idx[:, None, :]).reshape(B * K)
    cell = gx * GY + gy

    mem2d = memory.reshape(NCELL, H)
    ctx = _sc_gather(mem2d, idx25.reshape(NW, NCH, CH))
    out, upd = _dense(feature, hidden, ctx.reshape(B, K, H), weight_ih,
                      weight_hh, bias_ih, bias_hh, W_out, b_out)

    iota = jnp.arange(B, dtype=jnp.int32)
    last = jnp.full((NCELL,), -1, jnp.int32).at[cell].max(iota)
    win = last[cell]

    mem_ref = jax.new_ref(mem2d)
    _sc_scatter(upd, win.reshape(NW, NSC, CH), cell.reshape(NW, NSC, CH),
                mem_ref)
    new_memory = mem_ref[...].reshape(GX, GY, H)
    return (out, new_memory)


# speedup vs baseline: 3.5904x; 1.5332x over previous
import functools

import jax
import jax.numpy as jnp
from jax import lax
from jax.experimental import pallas as pl
from jax.experimental.pallas import tpu as pltpu
from jax.experimental.pallas import tpu_sc as plsc

B = 16384
IN = 130
H = 128
SW = 2
GRID = 600
GX = GRID + 3 * SW
GY = GRID + 3 * SW
NCELL = GX * GY
NOFF = 2 * SW + 1
K = NOFF * NOFF

NW = 32
SPW = B // NW
CE = 4
RPC = CE * K
NCHU = SPW // CE
CH = 128
NSC = SPW // CH
NV = H // 16

BB = 2048


def _gates_body(feat_ref, hid_ref, wih_ref, whh_ref, bih_ref, bhh_ref,
                ng_ref, sg_ref, ug_ref):
    dn = (((1,), (1,)), ((), ()))
    gi = lax.dot_general(feat_ref[...], wih_ref[...], dn,
                         preferred_element_type=jnp.float32) + bih_ref[...]
    gh = lax.dot_general(hid_ref[...], whh_ref[...], dn,
                         preferred_element_type=jnp.float32) + bhh_ref[...]
    resetgate = jax.nn.sigmoid(gi[:, 0:H] + gh[:, 0:H])
    ug_ref[...] = jax.nn.sigmoid(gi[:, H:2 * H] + gh[:, H:2 * H])
    sg_ref[...] = jax.nn.sigmoid(gi[:, 3 * H:4 * H] + gh[:, 3 * H:4 * H])
    ng_ref[...] = jnp.tanh(gi[:, 2 * H:3 * H] + resetgate * gh[:, 2 * H:3 * H])


def _gates(feature, hidden, wih, whh, bih, bhh):
    bs_b = pl.BlockSpec((BB, H), lambda i: (i, 0))
    full = lambda shape: pl.BlockSpec(shape, lambda i: tuple(0 for _ in shape))
    sds = jax.ShapeDtypeStruct((B, H), jnp.float32)
    return pl.pallas_call(
        _gates_body,
        grid=(B // BB,),
        in_specs=[bs_b, bs_b, full((4 * H, H)), full((4 * H, H)),
                  full((1, 4 * H)), full((1, 4 * H))],
        out_specs=[bs_b, bs_b, bs_b],
        out_shape=[sds, sds, sds],
        compiler_params=pltpu.CompilerParams(
            dimension_semantics=("arbitrary",)),
    )(feature, hidden, wih, whh, bih.reshape(1, 4 * H),
      bhh.reshape(1, 4 * H))


def _out_body(mix_ref, read_ref, ng_ref, sg_ref, ug_ref, hid_ref, wout_ref,
              bout_ref, out_ref, upd_ref):
    dn = (((1,), (1,)), ((), ()))
    ng = ng_ref[...]
    sg = sg_ref[...]
    wout = wout_ref[...]
    pre = (lax.dot_general(mix_ref[...], wout[:, 0:H], dn,
                           preferred_element_type=jnp.float32)
           + lax.dot_general(ng, wout[:, H:2 * H], dn,
                             preferred_element_type=jnp.float32)
           + bout_ref[...])
    atten_cs = jnp.tanh(pre)
    curr = ng + sg * atten_cs
    out = curr + ug_ref[...] * (hid_ref[...] - curr)
    out_ref[...] = out
    upd_ref[...] = sg * read_ref[...] + (1.0 - sg) * out


def _out(mix, read, ng, sg, ug, hidden, wout, bout):
    bs_b = pl.BlockSpec((BB, H), lambda i: (i, 0))
    full = lambda shape: pl.BlockSpec(shape, lambda i: tuple(0 for _ in shape))
    sds = jax.ShapeDtypeStruct((B, H), jnp.float32)
    return pl.pallas_call(
        _out_body,
        grid=(B // BB,),
        in_specs=[bs_b, bs_b, bs_b, bs_b, bs_b, bs_b, full((H, 2 * H)),
                  full((1, H))],
        out_specs=[bs_b, bs_b],
        out_shape=[sds, sds],
        compiler_params=pltpu.CompilerParams(
            dimension_semantics=("arbitrary",)),
    )(mix, read, ng, sg, ug, hidden, wout, bout.reshape(1, H))


_SC_MESH = plsc.VectorSubcoreMesh(core_axis_name="c", subcore_axis_name="s")


@functools.partial(
    pl.kernel,
    out_type=(jax.ShapeDtypeStruct((B, H), jnp.float32),
              jax.ShapeDtypeStruct((B, H), jnp.float32)),
    mesh=_SC_MESH,
    scratch_types=[
        pltpu.VMEM((NCHU, RPC), jnp.int32),
        pltpu.VMEM((2, RPC, H), jnp.float32),
        pltpu.VMEM((2, CE, H), jnp.float32),
        pltpu.VMEM((2, CE, H), jnp.float32),
        pltpu.VMEM((2, CE, H), jnp.float32),
        pltpu.SemaphoreType.DMA,
        pltpu.SemaphoreType.DMA,
        pltpu.SemaphoreType.DMA,
        pltpu.SemaphoreType.DMA,
        pltpu.SemaphoreType.DMA,
        pltpu.SemaphoreType.DMA,
        pltpu.SemaphoreType.DMA,
        pltpu.SemaphoreType.DMA,
    ],
)
def _sc_attend(mem_ref, idx_ref, ng_ref, mix_ref, read_ref,
               idx_v, rows_v, ng_v, mix_v, read_v,
               srA, srB, sgA, sgB, smA, smB, sdA, sdB):
    wid = lax.axis_index("s") * 2 + lax.axis_index("c")
    ebase = wid * SPW
    pltpu.sync_copy(idx_ref.at[wid], idx_v)
    sr = (srA, srB)
    sg = (sgA, sgB)
    sm = (smA, smB)
    sd = (sdA, sdB)

    def start_in(c, slot):
        pltpu.make_async_copy(mem_ref.at[idx_v.at[c]], rows_v.at[slot],
                              sr[slot]).start()
        pltpu.make_async_copy(ng_ref.at[pl.ds(ebase + c * CE, CE)],
                              ng_v.at[slot], sg[slot]).start()

    def wait_in(c, slot):
        pltpu.make_async_copy(mem_ref.at[idx_v.at[c]], rows_v.at[slot],
                              sr[slot]).wait()
        pltpu.make_async_copy(ng_ref.at[pl.ds(ebase + c * CE, CE)],
                              ng_v.at[slot], sg[slot]).wait()

    def start_out(c, slot):
        pltpu.make_async_copy(mix_v.at[slot],
                              mix_ref.at[pl.ds(ebase + c * CE, CE)],
                              sm[slot]).start()
        pltpu.make_async_copy(read_v.at[slot],
                              read_ref.at[pl.ds(ebase + c * CE, CE)],
                              sd[slot]).start()

    def wait_out(c, slot):
        pltpu.make_async_copy(mix_v.at[slot],
                              mix_ref.at[pl.ds(ebase + c * CE, CE)],
                              sm[slot]).wait()
        pltpu.make_async_copy(read_v.at[slot],
                              read_ref.at[pl.ds(ebase + c * CE, CE)],
                              sd[slot]).wait()

    zeros16 = jnp.zeros((16,), jnp.float32)
    ninf = jnp.float32(-jnp.inf)
    iota16 = lax.iota(jnp.int32, 16)
    shuf = [iota16 ^ sh for sh in (8, 4, 2, 1)]

    def bf_sum(x):
        for ix in shuf:
            x = x + x.at[ix].get(mode="promise_in_bounds")
        return x

    def bf_max(x):
        for ix in shuf:
            x = jnp.maximum(x, x.at[ix].get(mode="promise_in_bounds"))
        return x

    def compute(slot):
        for e in range(CE):
            ngv = [ng_v[slot, e, pl.ds(16 * v, 16)] for v in range(NV)]

            def dotbody(k, carry):
                a0, a1 = carry
                r = e * K + k
                acc = ngv[0] * rows_v[slot, r, pl.ds(0, 16)]
                for v in range(1, NV):
                    acc = acc + ngv[v] * rows_v[slot, r, pl.ds(16 * v, 16)]
                ak = bf_sum(acc)
                a0 = jnp.where(iota16 == k, ak, a0)
                a1 = jnp.where(iota16 == k - 16, ak, a1)
                return a0, a1

            a0, a1 = lax.fori_loop(0, K, dotbody, (zeros16, zeros16))
            z0 = a0 == 0.0
            z1 = a1 == 0.0
            m = bf_max(jnp.maximum(jnp.where(z0, ninf, a0),
                                   jnp.where(z1, ninf, a1)))
            m_safe = jnp.where(m == ninf, 0.0, m)
            e0 = jnp.where(z0, 0.0, jnp.exp(a0 - m_safe))
            e1 = jnp.where(z1, 0.0, jnp.exp(a1 - m_safe))
            s = bf_sum(e0 + e1)
            sinv = jnp.where(s > 0.0, 1.0 / s, 0.0)
            attn0 = e0 * sinv
            attn1 = e1 * sinv

            init = tuple(zeros16 for _ in range(NV))

            def mixbody(k, acc):
                r = e * K + k
                i0 = jnp.full((16,), jnp.minimum(k, 15), jnp.int32)
                i1 = jnp.full((16,), jnp.maximum(k - 16, 0), jnp.int32)
                w = jnp.where(k < 16,
                              attn0.at[i0].get(mode="promise_in_bounds"),
                              attn1.at[i1].get(mode="promise_in_bounds"))
                return tuple(acc[v] + w * rows_v[slot, r, pl.ds(16 * v, 16)]
                             for v in range(NV))

            acc = lax.fori_loop(0, K, mixbody, init)
            for v in range(NV):
                mix_v[slot, e, pl.ds(16 * v, 16)] = acc[v]
                read_v[slot, e, pl.ds(16 * v, 16)] = \
                    rows_v[slot, e * K + K // 2, pl.ds(16 * v, 16)]

    start_in(0, 0)
    start_in(1, 1)

    @pl.loop(0, NCHU, step=2)
    def _(g):
        for b in range(2):
            c = g + b

            @pl.when(c >= 2)
            def _():
                wait_out(c - 2, b)

            wait_in(c, b)
            compute(b)
            start_out(c, b)

            @pl.when(c + 2 < NCHU)
            def _():
                start_in(c + 2, b)

    wait_out(NCHU - 2, 0)
    wait_out(NCHU - 1, 1)


@functools.partial(
    pl.kernel,
    out_type=(),
    mesh=_SC_MESH,
    scratch_types=[
        pltpu.VMEM((NSC, CH), jnp.int32),
        pltpu.VMEM((NSC, CH), jnp.int32),
        pltpu.VMEM((CH, H), jnp.float32),
        pltpu.SemaphoreType.DMA,
    ],
)
def _sc_scatter(upd_ref, win_ref, cell_ref, mem_ref, win_v, cell_v, rows_v,
                sem):
    wid = lax.axis_index("s") * 2 + lax.axis_index("c")
    pltpu.sync_copy(win_ref.at[wid], win_v)
    pltpu.sync_copy(cell_ref.at[wid], cell_v)
    for j in range(NSC):
        cp = pltpu.make_async_copy(upd_ref.at[win_v.at[j]], rows_v, sem)
        cp.start()
        cp.wait()
        pltpu.sync_copy(rows_v, mem_ref.at[cell_v.at[j]])


def kernel(input_tensor, hidden, memory, weight_ih, weight_hh, bias_ih,
           bias_hh, W_out, b_out):
    feature = input_tensor[:, :H]
    coords = input_tensor[:, H:].astype(jnp.int32) + SW
    gx = jnp.clip(coords[:, 0], 0, GX - 1)
    gy = jnp.clip(coords[:, 1], 0, GY - 1)
    offs = jnp.arange(-SW, SW + 1, dtype=jnp.int32)
    x_idx = jnp.clip(gx[:, None] + offs[None, :], 0, GX - 1)
    y_idx = jnp.clip(gy[:, None] + offs[None, :], 0, GY - 1)
    idx25 = (x_idx[:, :, None] * GY + y_idx[:, None, :]).reshape(B * K)
    cell = gx * GY + gy

    mem2d = memory.reshape(NCELL, H)
    ng, sg, ug = _gates(feature, hidden, weight_ih, weight_hh, bias_ih,
                        bias_hh)
    mix, read = _sc_attend(mem2d, idx25.reshape(NW, NCHU, RPC), ng)
    out, upd = _out(mix, read, ng, sg, ug, hidden, W_out, b_out)

    iota = jnp.arange(B, dtype=jnp.int32)
    last = jnp.full((NCELL,), -1, jnp.int32).at[cell].max(iota)
    win = last[cell]

    mem_ref = jax.new_ref(mem2d)
    _sc_scatter(upd, win.reshape(NW, NSC, CH), cell.reshape(NW, NSC, CH),
                mem_ref)
    new_memory = mem_ref[...].reshape(GX, GY, H)
    return (out, new_memory)
